# Initial kernel scaffold; baseline (speedup 1.0000x reference)
import functools

import jax
import jax.numpy as jnp
from jax import lax
from jax.experimental import pallas as pl
from jax.experimental.pallas import tpu as pltpu
from jax.experimental.pallas import tpu_sc as plsc

N = 10000
E = 320000
NC = 2
NS = 16
NW = NC * NS
E_PER_TILE = E // NW
CHUNK = 80
N_CHUNKS = E_PER_TILE // CHUNK
ROWS_PER_SUB = N // NS
ZROWS = 125


def _scatter_add_sc(h, src, dst, D):
    mesh = plsc.VectorSubcoreMesh(core_axis_name="c", subcore_axis_name="s")

    @functools.partial(
        pl.kernel,
        out_type=jax.ShapeDtypeStruct((NC, N, D), jnp.float32),
        mesh=mesh,
        scratch_types=[
            pltpu.VMEM((CHUNK,), jnp.int32),
            pltpu.VMEM((CHUNK,), jnp.int32),
            pltpu.VMEM((CHUNK, D), jnp.float32),
            pltpu.VMEM((ZROWS, D), jnp.float32),
            pltpu.VMEM_SHARED((N, D), jnp.float32),
            pltpu.SemaphoreType.DMA,
        ],
    )
    def k(h_hbm, src_hbm, dst_hbm, out_hbm, src_v, dst_v, rows_v, zero_v,
          acc, sem):
        c = lax.axis_index("c")
        s = lax.axis_index("s")
        wid = c * NS + s

        zvec = jnp.zeros((16,), jnp.float32)

        @pl.loop(0, ZROWS)
        def _(r):
            for cc in range(D // 16):
                zero_v[r, pl.ds(cc * 16, 16)] = zvec

        @pl.loop(0, ROWS_PER_SUB // ZROWS)
        def _(j):
            pltpu.sync_copy(zero_v, acc.at[pl.ds(s * ROWS_PER_SUB + j * ZROWS,
                                                 ZROWS)])

        plsc.subcore_barrier()

        base0 = wid * E_PER_TILE

        @pl.loop(0, N_CHUNKS)
        def _(j):
            base = base0 + j * CHUNK
            pltpu.sync_copy(src_hbm.at[pl.ds(base, CHUNK)], src_v)
            pltpu.sync_copy(dst_hbm.at[pl.ds(base, CHUNK)], dst_v)
            pltpu.async_copy(h_hbm.at[src_v], rows_v, sem).wait()
            pltpu.sync_copy(rows_v, acc.at[dst_v], add=True)

        plsc.subcore_barrier()

        @pl.loop(0, ROWS_PER_SUB // ZROWS)
        def _(j):
            r0 = s * ROWS_PER_SUB + j * ZROWS
            pltpu.sync_copy(acc.at[pl.ds(r0, ZROWS)], zero_v)
            pltpu.sync_copy(zero_v, out_hbm.at[c].at[pl.ds(r0, ZROWS)])

    return k(h, src, dst)


def _proj_body(ix_ref, wp_ref, bp_ref, w1_ref, o_ref):
    wf = lax.dot_general(wp_ref[...], w1_ref[...], (((0,), (0,)), ((), ())),
                         preferred_element_type=jnp.float32)
    bf = lax.dot_general(bp_ref[...], w1_ref[...], (((1,), (0,)), ((), ())),
                         preferred_element_type=jnp.float32)
    o_ref[...] = lax.dot_general(ix_ref[...], wf, (((1,), (0,)), ((), ())),
                                 preferred_element_type=jnp.float32) + bf


def _layer2_body(p0_ref, p1_ref, w2_ref, o_ref):
    x1 = jnp.maximum(p0_ref[...] + p1_ref[...], 0.0)
    o_ref[...] = lax.dot_general(x1, w2_ref[...], (((1,), (0,)), ((), ())),
                                 preferred_element_type=jnp.float32)


def _sum_body(q0_ref, q1_ref, o_ref):
    o_ref[...] = q0_ref[...] + q1_ref[...]


def kernel(inputx, adj, nums, Wp, bp, W1, W2):
    del nums
    src = adj[0]
    dst = adj[1]
    nfeat = W1.shape[0]
    nhid = W1.shape[1]
    nclass = W2.shape[1]

    h1 = pl.pallas_call(
        _proj_body,
        out_shape=jax.ShapeDtypeStruct((N, nhid), jnp.float32),
    )(inputx, Wp, bp.reshape(1, nfeat), W1)

    p = _scatter_add_sc(h1, src, dst, nhid)

    blk = 1000
    h2 = pl.pallas_call(
        _layer2_body,
        grid=(N // blk,),
        in_specs=[
            pl.BlockSpec((blk, nhid), lambda i: (i, 0)),
            pl.BlockSpec((blk, nhid), lambda i: (i, 0)),
            pl.BlockSpec((nhid, nclass), lambda i: (0, 0)),
        ],
        out_specs=pl.BlockSpec((blk, nclass), lambda i: (i, 0)),
        out_shape=jax.ShapeDtypeStruct((N, nclass), jnp.float32),
    )(p[0], p[1], W2)

    q = _scatter_add_sc(h2, src, dst, nclass)

    out = pl.pallas_call(
        _sum_body,
        out_shape=jax.ShapeDtypeStruct((N, nclass), jnp.float32),
    )(q[0], q[1])
    return out


# speedup vs baseline: 4.6705x; 4.6705x over previous
import functools

import jax
import jax.numpy as jnp
from jax import lax
from jax.experimental import pallas as pl
from jax.experimental.pallas import tpu as pltpu
from jax.experimental.pallas import tpu_sc as plsc

N = 10000
E = 320000
NC = 2
NS = 16
NW = NC * NS
E_PER_TILE = E // NW
CHUNK = 80
N_CHUNKS = E_PER_TILE // CHUNK
ZROWS = 80
N_ROWBLK = N // ZROWS


def _scatter_add_sc(h, src, dst, D):
    mesh = plsc.VectorSubcoreMesh(core_axis_name="c", subcore_axis_name="s")

    @functools.partial(
        pl.kernel,
        out_type=jax.ShapeDtypeStruct((NC, N, D), jnp.float32),
        mesh=mesh,
        scratch_types=[
            pltpu.VMEM((CHUNK,), jnp.int32),
            pltpu.VMEM((CHUNK,), jnp.int32),
            pltpu.VMEM((CHUNK, D), jnp.float32),
            pltpu.VMEM((ZROWS, D), jnp.float32),
            pltpu.VMEM_SHARED((N, D), jnp.float32),
            pltpu.SemaphoreType.DMA,
        ],
        compiler_params=pltpu.CompilerParams(use_tc_tiling_on_sc=False),
    )
    def k(h_hbm, src_hbm, dst_hbm, out_hbm, src_v, dst_v, rows_v, zero_v,
          acc, sem):
        c = lax.axis_index("c")
        s = lax.axis_index("s")
        wid = c * NS + s

        zvec = jnp.zeros((16,), jnp.float32)

        @pl.loop(0, ZROWS)
        def _(r):
            for cc in range(D // 16):
                zero_v[r, pl.ds(cc * 16, 16)] = zvec

        @pl.loop(s, N_ROWBLK, step=NS)
        def _(j):
            pltpu.sync_copy(zero_v, acc.at[pl.ds(j * ZROWS, ZROWS)])

        plsc.subcore_barrier()

        base0 = wid * E_PER_TILE

        @pl.loop(0, N_CHUNKS)
        def _(j):
            base = base0 + j * CHUNK
            pltpu.sync_copy(src_hbm.at[pl.ds(base, CHUNK)], src_v)
            pltpu.sync_copy(dst_hbm.at[pl.ds(base, CHUNK)], dst_v)
            pltpu.async_copy(h_hbm.at[src_v], rows_v, sem).wait()
            pltpu.sync_copy(rows_v, acc.at[dst_v], add=True)

        plsc.subcore_barrier()

        @pl.loop(s, N_ROWBLK, step=NS)
        def _(j):
            r0 = j * ZROWS
            pltpu.sync_copy(acc.at[pl.ds(r0, ZROWS)], zero_v)
            pltpu.sync_copy(zero_v, out_hbm.at[c].at[pl.ds(r0, ZROWS)])

    return k(h, src, dst)


def _proj_body(ix_ref, wp_ref, bp_ref, w1_ref, o_ref):
    wf = lax.dot_general(wp_ref[...], w1_ref[...], (((0,), (0,)), ((), ())),
                         preferred_element_type=jnp.float32)
    bf = lax.dot_general(bp_ref[...], w1_ref[...], (((1,), (0,)), ((), ())),
                         preferred_element_type=jnp.float32)
    o_ref[...] = lax.dot_general(ix_ref[...], wf, (((1,), (0,)), ((), ())),
                                 preferred_element_type=jnp.float32) + bf


def _layer2_body(p0_ref, p1_ref, w2_ref, o_ref):
    x1 = jnp.maximum(p0_ref[...] + p1_ref[...], 0.0)
    o_ref[...] = lax.dot_general(x1, w2_ref[...], (((1,), (0,)), ((), ())),
                                 preferred_element_type=jnp.float32)


def _sum_body(q0_ref, q1_ref, o_ref):
    o_ref[...] = q0_ref[...] + q1_ref[...]


def kernel(inputx, adj, nums, Wp, bp, W1, W2):
    del nums
    src = adj[0]
    dst = adj[1]
    nfeat = W1.shape[0]
    nhid = W1.shape[1]
    nclass = W2.shape[1]

    h1 = pl.pallas_call(
        _proj_body,
        out_shape=jax.ShapeDtypeStruct((N, nhid), jnp.float32),
    )(inputx, Wp, bp.reshape(1, nfeat), W1)

    p = _scatter_add_sc(h1, src, dst, nhid)

    blk = 1000
    h2 = pl.pallas_call(
        _layer2_body,
        grid=(N // blk,),
        in_specs=[
            pl.BlockSpec((blk, nhid), lambda i: (i, 0)),
            pl.BlockSpec((blk, nhid), lambda i: (i, 0)),
            pl.BlockSpec((nhid, nclass), lambda i: (0, 0)),
        ],
        out_specs=pl.BlockSpec((blk, nclass), lambda i: (i, 0)),
        out_shape=jax.ShapeDtypeStruct((N, nclass), jnp.float32),
    )(p[0], p[1], W2)

    q = _scatter_add_sc(h2, src, dst, nclass)

    out = pl.pallas_call(
        _sum_body,
        out_shape=jax.ShapeDtypeStruct((N, nclass), jnp.float32),
    )(q[0], q[1])
    return out


# speedup vs baseline: 11.7278x; 2.5110x over previous
import functools

import jax
import jax.numpy as jnp
from jax import lax
from jax.experimental import pallas as pl
from jax.experimental.pallas import tpu as pltpu
from jax.experimental.pallas import tpu_sc as plsc

N = 10000
E = 320000
NC = 2
NS = 16
NW = NC * NS
E_PER_TILE = E // NW
CHUNK = 125
N_CHUNKS = E_PER_TILE // CHUNK
NBUF = 2
NIDX = 4
ZROWS = 80
N_ROWBLK = N // ZROWS


def _scatter_add_sc(h, idx4, D):
    mesh = plsc.VectorSubcoreMesh(core_axis_name="c", subcore_axis_name="s")

    @functools.partial(
        pl.kernel,
        out_type=jax.ShapeDtypeStruct((NC, N, D), jnp.float32),
        mesh=mesh,
        scratch_types=[
            [pltpu.VMEM((2, CHUNK), jnp.int32) for _ in range(NIDX)],
            [pltpu.VMEM((CHUNK, D), jnp.float32) for _ in range(NBUF)],
            [pltpu.SemaphoreType.DMA for _ in range(NIDX)],
            [pltpu.SemaphoreType.DMA for _ in range(NBUF)],
            [pltpu.SemaphoreType.DMA for _ in range(NBUF)],
            pltpu.VMEM((ZROWS, D), jnp.float32),
            pltpu.VMEM_SHARED((N, D), jnp.float32),
        ],
        compiler_params=pltpu.CompilerParams(use_tc_tiling_on_sc=False),
    )
    def k(h_hbm, idx_hbm, out_hbm, islots, rows, isems, gsems, ssems,
          zero_v, acc):
        c = lax.axis_index("c")
        s = lax.axis_index("s")
        wid = c * NS + s

        zvec = jnp.zeros((16,), jnp.float32)

        @pl.loop(0, ZROWS)
        def _(r):
            for cc in range(D // 16):
                zero_v[r, pl.ds(cc * 16, 16)] = zvec

        @pl.loop(s, N_ROWBLK, step=NS)
        def _(j):
            pltpu.sync_copy(zero_v, acc.at[pl.ds(j * ZROWS, ZROWS)])

        plsc.subcore_barrier()

        def load_idx(j, sl):
            pltpu.async_copy(idx_hbm.at[wid].at[j], islots[sl], isems[sl])

        def wait_idx(sl):
            pltpu.make_async_copy(idx_hbm.at[wid].at[0], islots[sl],
                                  isems[sl]).wait()

        def start_gather(sl, b):
            pltpu.async_copy(h_hbm.at[islots[sl].at[0]], rows[b], gsems[b])

        def wait_gather(sl, b):
            pltpu.make_async_copy(h_hbm.at[islots[sl].at[0]], rows[b],
                                  gsems[b]).wait()

        def start_scatter(sl, b):
            pltpu.async_copy(rows[b], acc.at[islots[sl].at[1]], ssems[b],
                             add=True)

        def wait_scatter(sl, b):
            pltpu.make_async_copy(rows[b], acc.at[islots[sl].at[1]],
                                  ssems[b]).wait()

        load_idx(0, 0)
        load_idx(1, 1)
        load_idx(2, 2)
        wait_idx(0)
        start_gather(0, 0)

        @pl.loop(0, N_CHUNKS // NIDX)
        def _(g):
            j0 = g * NIDX
            for q in range(NIDX):
                j = j0 + q
                b = q % NBUF
                bn = (b + 1) % NBUF
                qn = (q + 1) % NIDX
                qp = (q + 3) % NIDX

                @pl.when(j >= 1)
                def _():
                    wait_scatter(qp, bn)

                @pl.when(j + 1 < N_CHUNKS)
                def _():
                    wait_idx(qn)
                    start_gather(qn, bn)

                wait_gather(q, b)

                @pl.when(j + 3 < N_CHUNKS)
                def _():
                    load_idx(j + 3, qp)

                start_scatter(q, b)

        wait_scatter((N_CHUNKS - 1) % NIDX, (N_CHUNKS - 1) % NBUF)

        plsc.subcore_barrier()

        @pl.loop(s, N_ROWBLK, step=NS)
        def _(j):
            r0 = j * ZROWS
            pltpu.sync_copy(acc.at[pl.ds(r0, ZROWS)], zero_v)
            pltpu.sync_copy(zero_v, out_hbm.at[c].at[pl.ds(r0, ZROWS)])

    return k(h, idx4)


def _proj_body(ix_ref, wp_ref, bp_ref, w1_ref, o_ref):
    wf = lax.dot_general(wp_ref[...], w1_ref[...], (((0,), (0,)), ((), ())),
                         preferred_element_type=jnp.float32)
    bf = lax.dot_general(bp_ref[...], w1_ref[...], (((1,), (0,)), ((), ())),
                         preferred_element_type=jnp.float32)
    o_ref[...] = lax.dot_general(ix_ref[...], wf, (((1,), (0,)), ((), ())),
                                 preferred_element_type=jnp.float32) + bf


def _layer2_body(p0_ref, p1_ref, w2_ref, o_ref):
    x1 = jnp.maximum(p0_ref[...] + p1_ref[...], 0.0)
    o_ref[...] = lax.dot_general(x1, w2_ref[...], (((1,), (0,)), ((), ())),
                                 preferred_element_type=jnp.float32)


def _sum_body(q0_ref, q1_ref, o_ref):
    o_ref[...] = q0_ref[...] + q1_ref[...]


def kernel(inputx, adj, nums, Wp, bp, W1, W2):
    del nums
    idx4 = jnp.transpose(adj.reshape(2, NW, N_CHUNKS, CHUNK), (1, 2, 0, 3))
    nfeat = W1.shape[0]
    nhid = W1.shape[1]
    nclass = W2.shape[1]

    h1 = pl.pallas_call(
        _proj_body,
        out_shape=jax.ShapeDtypeStruct((N, nhid), jnp.float32),
    )(inputx, Wp, bp.reshape(1, nfeat), W1)

    p = _scatter_add_sc(h1, idx4, nhid)

    blk = 1000
    h2 = pl.pallas_call(
        _layer2_body,
        grid=(N // blk,),
        in_specs=[
            pl.BlockSpec((blk, nhid), lambda i: (i, 0)),
            pl.BlockSpec((blk, nhid), lambda i: (i, 0)),
            pl.BlockSpec((nhid, nclass), lambda i: (0, 0)),
        ],
        out_specs=pl.BlockSpec((blk, nclass), lambda i: (i, 0)),
        out_shape=jax.ShapeDtypeStruct((N, nclass), jnp.float32),
    )(p[0], p[1], W2)

    q = _scatter_add_sc(h2, idx4, nclass)

    out = pl.pallas_call(
        _sum_body,
        out_shape=jax.ShapeDtypeStruct((N, nclass), jnp.float32),
    )(q[0], q[1])
    return out


# speedup vs baseline: 12.5495x; 1.0701x over previous
import functools

import jax
import jax.numpy as jnp
from jax import lax
from jax.experimental import pallas as pl
from jax.experimental.pallas import tpu as pltpu
from jax.experimental.pallas import tpu_sc as plsc

N = 10000
E = 320000
NC = 2
NS = 16
NW = NC * NS
E_PER_TILE = E // NW
CHUNK = 125
N_CHUNKS = E_PER_TILE // CHUNK
NBUF = 2
NIDX = 4
ZROWS = 80
N_ROWBLK = N // ZROWS


def _scatter_add_sc(h, idx, D):
    mesh = plsc.VectorSubcoreMesh(core_axis_name="c", subcore_axis_name="s")

    @functools.partial(
        pl.kernel,
        out_type=jax.ShapeDtypeStruct((NC, N, D), jnp.float32),
        mesh=mesh,
        scratch_types=[
            [pltpu.VMEM((CHUNK,), jnp.int32) for _ in range(NIDX)],
            [pltpu.VMEM((CHUNK,), jnp.int32) for _ in range(NIDX)],
            [pltpu.VMEM((CHUNK, D), jnp.float32) for _ in range(NBUF)],
            [pltpu.SemaphoreType.DMA for _ in range(NIDX)],
            [pltpu.SemaphoreType.DMA for _ in range(NIDX)],
            [pltpu.SemaphoreType.DMA for _ in range(NBUF)],
            [pltpu.SemaphoreType.DMA for _ in range(NBUF)],
            pltpu.VMEM((ZROWS, D), jnp.float32),
            pltpu.VMEM_SHARED((N, D), jnp.float32),
        ],
        compiler_params=pltpu.CompilerParams(use_tc_tiling_on_sc=False),
    )
    def k(h_hbm, idx_hbm, out_hbm, srcs, dsts, rows, s_sems, d_sems, gsems,
          ssems, zero_v, acc):
        c = lax.axis_index("c")
        s = lax.axis_index("s")
        wid = c * NS + s

        zvec = jnp.zeros((16,), jnp.float32)

        @pl.loop(0, ZROWS)
        def _(r):
            for cc in range(D // 16):
                zero_v[r, pl.ds(cc * 16, 16)] = zvec

        @pl.loop(s, N_ROWBLK, step=NS)
        def _(j):
            pltpu.sync_copy(zero_v, acc.at[pl.ds(j * ZROWS, ZROWS)])

        plsc.subcore_barrier()

        def load_idx(j, sl):
            pltpu.async_copy(idx_hbm.at[0].at[wid].at[j], srcs[sl],
                             s_sems[sl])
            pltpu.async_copy(idx_hbm.at[1].at[wid].at[j], dsts[sl],
                             d_sems[sl])

        def wait_idx(sl):
            pltpu.make_async_copy(idx_hbm.at[0].at[wid].at[0], srcs[sl],
                                  s_sems[sl]).wait()
            pltpu.make_async_copy(idx_hbm.at[1].at[wid].at[0], dsts[sl],
                                  d_sems[sl]).wait()

        def start_gather(sl, b):
            pltpu.async_copy(h_hbm.at[srcs[sl]], rows[b], gsems[b])

        def wait_gather(sl, b):
            pltpu.make_async_copy(h_hbm.at[srcs[sl]], rows[b],
                                  gsems[b]).wait()

        def start_scatter(sl, b):
            pltpu.async_copy(rows[b], acc.at[dsts[sl]], ssems[b], add=True)

        def wait_scatter(sl, b):
            pltpu.make_async_copy(rows[b], acc.at[dsts[sl]],
                                  ssems[b]).wait()

        load_idx(0, 0)
        load_idx(1, 1)
        load_idx(2, 2)
        wait_idx(0)
        start_gather(0, 0)

        @pl.loop(0, N_CHUNKS // NIDX)
        def _(g):
            j0 = g * NIDX
            for q in range(NIDX):
                j = j0 + q
                b = q % NBUF
                bn = (b + 1) % NBUF
                qn = (q + 1) % NIDX
                qp = (q + 3) % NIDX

                @pl.when(j >= 1)
                def _():
                    wait_scatter(qp, bn)

                @pl.when(j + 1 < N_CHUNKS)
                def _():
                    wait_idx(qn)
                    start_gather(qn, bn)

                wait_gather(q, b)

                @pl.when(j + 3 < N_CHUNKS)
                def _():
                    load_idx(j + 3, qp)

                start_scatter(q, b)

        wait_scatter((N_CHUNKS - 1) % NIDX, (N_CHUNKS - 1) % NBUF)

        plsc.subcore_barrier()

        @pl.loop(s, N_ROWBLK, step=NS)
        def _(j):
            r0 = j * ZROWS
            pltpu.sync_copy(acc.at[pl.ds(r0, ZROWS)], zero_v)
            pltpu.sync_copy(zero_v, out_hbm.at[c].at[pl.ds(r0, ZROWS)])

    return k(h, idx)


def _proj_body(ix_ref, wp_ref, bp_ref, w1_ref, o_ref):
    wf = lax.dot_general(wp_ref[...], w1_ref[...], (((0,), (0,)), ((), ())),
                         preferred_element_type=jnp.float32)
    bf = lax.dot_general(bp_ref[...], w1_ref[...], (((1,), (0,)), ((), ())),
                         preferred_element_type=jnp.float32)
    o_ref[...] = lax.dot_general(ix_ref[...], wf, (((1,), (0,)), ((), ())),
                                 preferred_element_type=jnp.float32) + bf


def _layer2_body(p_ref, w2_ref, o_ref):
    x1 = jnp.maximum(p_ref[0] + p_ref[1], 0.0)
    o_ref[...] = lax.dot_general(x1, w2_ref[...], (((1,), (0,)), ((), ())),
                                 preferred_element_type=jnp.float32)


def _sum_body(q_ref, o_ref):
    o_ref[...] = q_ref[0] + q_ref[1]


def kernel(inputx, adj, nums, Wp, bp, W1, W2):
    del nums
    idx = adj.reshape(2, NW, N_CHUNKS, CHUNK)
    nfeat = W1.shape[0]
    nhid = W1.shape[1]
    nclass = W2.shape[1]

    h1 = pl.pallas_call(
        _proj_body,
        out_shape=jax.ShapeDtypeStruct((N, nhid), jnp.float32),
    )(inputx, Wp, bp.reshape(1, nfeat), W1)

    p = _scatter_add_sc(h1, idx, nhid)

    blk = 1000
    h2 = pl.pallas_call(
        _layer2_body,
        grid=(N // blk,),
        in_specs=[
            pl.BlockSpec((NC, blk, nhid), lambda i: (0, i, 0)),
            pl.BlockSpec((nhid, nclass), lambda i: (0, 0)),
        ],
        out_specs=pl.BlockSpec((blk, nclass), lambda i: (i, 0)),
        out_shape=jax.ShapeDtypeStruct((N, nclass), jnp.float32),
    )(p, W2)

    q = _scatter_add_sc(h2, idx, nclass)

    out = pl.pallas_call(
        _sum_body,
        out_shape=jax.ShapeDtypeStruct((N, nclass), jnp.float32),
    )(q)
    return out
